# baseline (speedup 1.0000x reference)
import functools

import jax
import jax.numpy as jnp
from jax import lax
from jax.experimental import pallas as pl
from jax.experimental.pallas import tpu as pltpu
from jax.experimental.pallas import tpu_sc as plsc

_K = 32
_BR = 256


def _topk_body(xr_ref, yr_ref, sqr_ref, xc_ref, yc_ref, sqc_ref, out_ref):
    i = pl.program_id(0)
    br = xr_ref.shape[0]
    n = xc_ref.shape[1]
    xr = xr_ref[...]
    yr = yr_ref[...]
    sqr = sqr_ref[...]
    xc = xc_ref[...]
    yc = yc_ref[...]
    sqc = sqc_ref[...]

    xrb = xr.astype(jnp.bfloat16).astype(jnp.float32)
    yrb = yr.astype(jnp.bfloat16).astype(jnp.float32)
    xcb = xc.astype(jnp.bfloat16).astype(jnp.float32)
    ycb = yc.astype(jnp.bfloat16).astype(jnp.float32)

    cross = xrb * xcb + yrb * ycb
    d2 = (sqr + sqc) - 2.0 * cross
    neg = -d2

    colid = jax.lax.broadcasted_iota(jnp.int32, (1, n), 1)
    row_ids = i * br + jax.lax.broadcasted_iota(jnp.int32, (br, 1), 0)
    neg = jnp.where(colid == row_ids, -jnp.inf, neg)

    sels = []
    for _ in range(_K):
        g = jnp.max(neg, axis=1, keepdims=True)
        cand = jnp.where(neg == g, colid, n)
        sel = jnp.min(cand, axis=1, keepdims=True)
        sels.append(sel)
        neg = jnp.where(colid == sel, -jnp.inf, neg)
    out_ref[...] = jnp.concatenate(sels, axis=1)


def _topk_indices(p):
    n = p.shape[0]
    x = p[:, 0]
    y = p[:, 1]
    sq = jnp.sum(p[:, :2] * p[:, :2], axis=1)
    xr = x.reshape(n, 1)
    yr = y.reshape(n, 1)
    sqr = sq.reshape(n, 1)
    xc = x.reshape(1, n)
    yc = y.reshape(1, n)
    sqc = sq.reshape(1, n)
    grid = n // _BR
    row_spec = pl.BlockSpec((_BR, 1), lambda i: (i, 0))
    col_spec = pl.BlockSpec((1, n), lambda i: (0, 0))
    return pl.pallas_call(
        _topk_body,
        grid=(grid,),
        in_specs=[row_spec, row_spec, row_spec, col_spec, col_spec, col_spec],
        out_specs=pl.BlockSpec((_BR, _K), lambda i: (i, 0)),
        out_shape=jax.ShapeDtypeStruct((n, _K), jnp.int32),
        compiler_params=pltpu.CompilerParams(
            dimension_semantics=("parallel",),
        ),
    )(xr, yr, sqr, xc, yc, sqc)


def _disp_gather(p, idx):
    n, k = idx.shape
    info = plsc.get_sparse_core_info()
    nc = info.num_cores
    nw = nc * info.num_subcores
    rw = n // nw
    x = p[:, 0]
    y = p[:, 1]
    mesh = plsc.VectorSubcoreMesh(core_axis_name="c", subcore_axis_name="s")

    @functools.partial(
        pl.kernel,
        mesh=mesh,
        compiler_params=pltpu.CompilerParams(
            use_tc_tiling_on_sc=False, needs_layout_passes=False
        ),
        out_type=jax.ShapeDtypeStruct((n * k * 2,), jnp.float32),
        scratch_types=[
            pltpu.VMEM((n,), jnp.float32),
            pltpu.VMEM((n,), jnp.float32),
            pltpu.VMEM((rw * k,), jnp.int32),
            pltpu.VMEM((rw * k * 2,), jnp.float32),
        ],
    )
    def sc_gather(x_hbm, y_hbm, idx_hbm, out_hbm, xv, yv, idxv, outv):
        wid = lax.axis_index("s") * nc + lax.axis_index("c")
        base = wid * rw
        pltpu.sync_copy(x_hbm, xv)
        pltpu.sync_copy(y_hbm, yv)
        pltpu.sync_copy(idx_hbm.at[pl.ds(base * k, rw * k)], idxv)
        iota16 = lax.iota(jnp.int32, 16)

        def row_body(r, carry):
            rvec = jnp.full((16,), base + r, jnp.int32)
            rx = plsc.load_gather(xv, [rvec])
            ry = plsc.load_gather(yv, [rvec])
            for v in range(k // 16):
                iv = idxv[pl.ds(r * k + v * 16, 16)]
                gx = plsc.load_gather(xv, [iv])
                gy = plsc.load_gather(yv, [iv])
                pos = r * (k * 2) + v * 32 + 2 * iota16
                plsc.store_scatter(outv, [pos], rx - gx)
                plsc.store_scatter(outv, [pos + 1], ry - gy)
            return carry

        lax.fori_loop(0, rw, row_body, 0)
        pltpu.sync_copy(outv, out_hbm.at[pl.ds(base * k * 2, rw * k * 2)])

    return sc_gather(x, y, idx.reshape(-1)).reshape(n, k, 2)


def kernel(p):
    idx = _topk_indices(p)
    return _disp_gather(p, idx)


# speedup vs baseline: 2.7852x; 2.7852x over previous
import functools

import jax
import jax.numpy as jnp
from jax import lax
from jax.experimental import pallas as pl
from jax.experimental.pallas import tpu as pltpu
from jax.experimental.pallas import tpu_sc as plsc

_K = 32
_BR = 256
_BR3 = 512
_NCH = 512
_CA = 32
_KCH = 48
_NCAND = _KCH * _CA


def _neg_dist_block(xr, yr, sqr, xc, yc, sqc):
    xrb = xr.astype(jnp.bfloat16).astype(jnp.float32)
    yrb = yr.astype(jnp.bfloat16).astype(jnp.float32)
    xcb = xc.astype(jnp.bfloat16).astype(jnp.float32)
    ycb = yc.astype(jnp.bfloat16).astype(jnp.float32)
    cross = xrb * xcb + yrb * ycb
    d2 = (sqr + sqc) - 2.0 * cross
    return -d2


def _stage1_body(xr_ref, yr_ref, sqr_ref, xc_ref, yc_ref, sqc_ref, out_ref):
    i = pl.program_id(0)
    br = xr_ref.shape[0]
    n = xc_ref.shape[1]
    neg = _neg_dist_block(xr_ref[...], yr_ref[...], sqr_ref[...],
                          xc_ref[...], yc_ref[...], sqc_ref[...])
    colid = lax.broadcasted_iota(jnp.int32, (1, n), 1)
    row_ids = i * br + lax.broadcasted_iota(jnp.int32, (br, 1), 0)
    neg = jnp.where(colid == row_ids, -jnp.inf, neg)

    cm = neg[:, 0:_NCH]
    for a in range(1, _CA):
        cm = jnp.maximum(cm, neg[:, a * _NCH:(a + 1) * _NCH])

    chid = lax.broadcasted_iota(jnp.int32, (1, _NCH), 1)
    sels = []
    for _ in range(_KCH):
        g = jnp.max(cm, axis=1, keepdims=True)
        sel = jnp.min(jnp.where(cm == g, chid, _NCH), axis=1, keepdims=True)
        sels.append(sel)
        cm = jnp.where(chid == sel, -jnp.inf, cm)
    out_ref[...] = jnp.concatenate(sels, axis=1)


def _top_chunks(p):
    n = p.shape[0]
    x = p[:, 0]
    y = p[:, 1]
    sq = jnp.sum(p[:, :2] * p[:, :2], axis=1)
    grid = n // _BR
    row_spec = pl.BlockSpec((_BR, 1), lambda i: (i, 0))
    col_spec = pl.BlockSpec((1, n), lambda i: (0, 0))
    return pl.pallas_call(
        _stage1_body,
        grid=(grid,),
        in_specs=[row_spec, row_spec, row_spec, col_spec, col_spec, col_spec],
        out_specs=pl.BlockSpec((_BR, _KCH), lambda i: (i, 0)),
        out_shape=jax.ShapeDtypeStruct((n, _KCH), jnp.int32),
        compiler_params=pltpu.CompilerParams(
            dimension_semantics=("parallel",),
        ),
    )(x.reshape(n, 1), y.reshape(n, 1), sq.reshape(n, 1),
      x.reshape(1, n), y.reshape(1, n), sq.reshape(1, n))


def _bf16_round_sc(v):
    bits = plsc.bitcast(v, jnp.int32)
    r = (bits + 0x7FFF + ((bits >> 16) & 1)) & jnp.int32(-65536)
    return plsc.bitcast(r, jnp.float32)


def _stage2_cand(p, chunks):
    n = p.shape[0]
    info = plsc.get_sparse_core_info()
    nc = info.num_cores
    nw = nc * info.num_subcores
    rw = n // nw
    rg = 32
    ngr = rw // rg
    x = p[:, 0]
    y = p[:, 1]
    sq = jnp.sum(p[:, :2] * p[:, :2], axis=1)
    mesh = plsc.VectorSubcoreMesh(core_axis_name="c", subcore_axis_name="s")

    @functools.partial(
        pl.kernel,
        mesh=mesh,
        compiler_params=pltpu.CompilerParams(
            use_tc_tiling_on_sc=False, needs_layout_passes=False
        ),
        out_type=jax.ShapeDtypeStruct((n * _NCAND,), jnp.float32),
        scratch_types=[
            pltpu.VMEM((n,), jnp.float32),
            pltpu.VMEM((n,), jnp.float32),
            pltpu.VMEM((n,), jnp.float32),
            pltpu.VMEM((rg * _KCH,), jnp.int32),
            pltpu.VMEM((rg * _NCAND,), jnp.float32),
        ],
    )
    def sc_cand(x_hbm, y_hbm, sq_hbm, ch_hbm, out_hbm, xv, yv, sqv, chv, outv):
        wid = lax.axis_index("s") * nc + lax.axis_index("c")
        base = wid * rw
        pltpu.sync_copy(x_hbm, xv)
        pltpu.sync_copy(y_hbm, yv)
        pltpu.sync_copy(sq_hbm, sqv)

        def group_body(g, carry):
            r0 = base + g * rg
            pltpu.sync_copy(ch_hbm.at[pl.ds(r0 * _KCH, rg * _KCH)], chv)

            def row_body(r, carry2):
                rid = jnp.full((16,), r0 + r, jnp.int32)
                rx = plsc.load_gather(xv, [rid])
                ry = plsc.load_gather(yv, [rid])
                rsq = plsc.load_gather(sqv, [rid])
                rxb = _bf16_round_sc(rx)
                ryb = _bf16_round_sc(ry)
                for h in range(_KCH // 16):
                    cv = chv[pl.ds(r * _KCH + h * 16, 16)]
                    for a in range(_CA):
                        cand = cv + _NCH * a
                        gx = plsc.load_gather(xv, [cand])
                        gy = plsc.load_gather(yv, [cand])
                        gsq = plsc.load_gather(sqv, [cand])
                        gxb = _bf16_round_sc(gx)
                        gyb = _bf16_round_sc(gy)
                        cross = rxb * gxb + ryb * gyb
                        d2 = (rsq + gsq) - 2.0 * cross
                        outv[pl.ds(r * _NCAND + a * _KCH + h * 16, 16)] = d2
                return carry2

            lax.fori_loop(0, rg, row_body, 0)
            pltpu.sync_copy(
                outv, out_hbm.at[pl.ds(r0 * _NCAND, rg * _NCAND)]
            )
            return carry

        lax.fori_loop(0, ngr, group_body, 0)

    return sc_cand(x, y, sq, chunks.reshape(-1))


def _stage3_body(d2_ref, cand_ref, out_ref):
    i = pl.program_id(0)
    br = d2_ref.shape[0]
    n_total = _NCH * _CA
    neg = -d2_ref[...]
    candid = cand_ref[...]
    row_ids = i * br + lax.broadcasted_iota(jnp.int32, (br, 1), 0)
    neg = jnp.where(candid == row_ids, -jnp.inf, neg)

    sels = []
    for _ in range(_K):
        g = jnp.max(neg, axis=1, keepdims=True)
        sel = jnp.min(jnp.where(neg == g, candid, n_total),
                      axis=1, keepdims=True)
        sels.append(sel)
        neg = jnp.where(candid == sel, -jnp.inf, neg)
    out_ref[...] = jnp.concatenate(sels, axis=1)


def _topk_of_candidates(d2c, chunks):
    n = chunks.shape[0]
    grid = n // _BR3
    cand = (chunks[:, None, :]
            + _NCH * jnp.arange(_CA, dtype=jnp.int32)[:, None]
            ).reshape(n, _NCAND)
    return pl.pallas_call(
        _stage3_body,
        grid=(grid,),
        in_specs=[
            pl.BlockSpec((_BR3, _NCAND), lambda i: (i, 0)),
            pl.BlockSpec((_BR3, _NCAND), lambda i: (i, 0)),
        ],
        out_specs=pl.BlockSpec((_BR3, _K), lambda i: (i, 0)),
        out_shape=jax.ShapeDtypeStruct((n, _K), jnp.int32),
        compiler_params=pltpu.CompilerParams(
            dimension_semantics=("parallel",),
        ),
    )(d2c.reshape(n, _NCAND), cand)


def _disp_gather(p, idx):
    n, k = idx.shape
    info = plsc.get_sparse_core_info()
    nc = info.num_cores
    nw = nc * info.num_subcores
    rw = n // nw
    x = p[:, 0]
    y = p[:, 1]
    mesh = plsc.VectorSubcoreMesh(core_axis_name="c", subcore_axis_name="s")

    @functools.partial(
        pl.kernel,
        mesh=mesh,
        compiler_params=pltpu.CompilerParams(
            use_tc_tiling_on_sc=False, needs_layout_passes=False
        ),
        out_type=jax.ShapeDtypeStruct((n * k * 2,), jnp.float32),
        scratch_types=[
            pltpu.VMEM((n,), jnp.float32),
            pltpu.VMEM((n,), jnp.float32),
            pltpu.VMEM((rw * k,), jnp.int32),
            pltpu.VMEM((rw * k * 2,), jnp.float32),
        ],
    )
    def sc_gather(x_hbm, y_hbm, idx_hbm, out_hbm, xv, yv, idxv, outv):
        wid = lax.axis_index("s") * nc + lax.axis_index("c")
        base = wid * rw
        pltpu.sync_copy(x_hbm, xv)
        pltpu.sync_copy(y_hbm, yv)
        pltpu.sync_copy(idx_hbm.at[pl.ds(base * k, rw * k)], idxv)
        iota16 = lax.iota(jnp.int32, 16)

        def row_body(r, carry):
            rvec = jnp.full((16,), base + r, jnp.int32)
            rx = plsc.load_gather(xv, [rvec])
            ry = plsc.load_gather(yv, [rvec])
            for v in range(k // 16):
                iv = idxv[pl.ds(r * k + v * 16, 16)]
                gx = plsc.load_gather(xv, [iv])
                gy = plsc.load_gather(yv, [iv])
                pos = r * (k * 2) + v * 32 + 2 * iota16
                plsc.store_scatter(outv, [pos], rx - gx)
                plsc.store_scatter(outv, [pos + 1], ry - gy)
            return carry

        lax.fori_loop(0, rw, row_body, 0)
        pltpu.sync_copy(outv, out_hbm.at[pl.ds(base * k * 2, rw * k * 2)])

    return sc_gather(x, y, idx.reshape(-1)).reshape(n, k, 2)


def kernel(p):
    chunks = _top_chunks(p)
    d2c = _stage2_cand(p, chunks)
    idx = _topk_of_candidates(d2c, chunks)
    return _disp_gather(p, idx)


# speedup vs baseline: 3.0803x; 1.1059x over previous
import functools

import jax
import jax.numpy as jnp
from jax import lax
from jax.experimental import pallas as pl
from jax.experimental.pallas import tpu as pltpu
from jax.experimental.pallas import tpu_sc as plsc

_K = 32
_BR = 256
_BR3 = 512
_NCH = 512
_CA = 32
_KCH = 48
_NCAND = _KCH * _CA


def _stage1_body(xr_ref, yr_ref, sqr_ref, xc_ref, yc_ref, sqc_ref, out_ref):
    n = xc_ref.shape[1]
    xrb = xr_ref[...].astype(jnp.bfloat16)
    yrb = yr_ref[...].astype(jnp.bfloat16)
    xcb = xc_ref[...].astype(jnp.bfloat16)
    ycb = yc_ref[...].astype(jnp.bfloat16)
    cross = jax.lax.dot_general(
        jnp.concatenate([xrb, yrb], axis=1),
        jnp.concatenate([xcb, ycb], axis=0),
        (((1,), (0,)), ((), ())),
        preferred_element_type=jnp.float32,
    )
    d2 = (sqr_ref[...] + sqc_ref[...]) - 2.0 * cross
    neg = -d2

    cm = neg[:, 0:_NCH]
    for a in range(1, _CA):
        cm = jnp.maximum(cm, neg[:, a * _NCH:(a + 1) * _NCH])

    chid = lax.broadcasted_iota(jnp.int32, (1, _NCH), 1)
    sels = []
    for _ in range(_KCH):
        g = jnp.max(cm, axis=1, keepdims=True)
        sel = jnp.min(jnp.where(cm == g, chid, _NCH), axis=1, keepdims=True)
        sels.append(sel)
        cm = jnp.where(chid == sel, -jnp.inf, cm)
    out_ref[...] = jnp.concatenate(sels, axis=1)


def _top_chunks(p):
    n = p.shape[0]
    x = p[:, 0]
    y = p[:, 1]
    sq = jnp.sum(p[:, :2] * p[:, :2], axis=1)
    grid = n // _BR
    row_spec = pl.BlockSpec((_BR, 1), lambda i: (i, 0))
    col_spec = pl.BlockSpec((1, n), lambda i: (0, 0))
    return pl.pallas_call(
        _stage1_body,
        grid=(grid,),
        in_specs=[row_spec, row_spec, row_spec, col_spec, col_spec, col_spec],
        out_specs=pl.BlockSpec((_BR, _KCH), lambda i: (i, 0)),
        out_shape=jax.ShapeDtypeStruct((n, _KCH), jnp.int32),
        compiler_params=pltpu.CompilerParams(
            dimension_semantics=("parallel",),
        ),
    )(x.reshape(n, 1), y.reshape(n, 1), sq.reshape(n, 1),
      x.reshape(1, n), y.reshape(1, n), sq.reshape(1, n))


def _bf16_round_sc(v):
    bits = plsc.bitcast(v, jnp.int32)
    r = (bits + 0x7FFF + ((bits >> 16) & 1)) & jnp.int32(-65536)
    return plsc.bitcast(r, jnp.float32)


def _stage2_cand(p, chunks):
    n = p.shape[0]
    info = plsc.get_sparse_core_info()
    nc = info.num_cores
    nw = nc * info.num_subcores
    rw = n // nw
    rg = 32
    ngr = rw // rg
    x = p[:, 0]
    y = p[:, 1]
    sq = jnp.sum(p[:, :2] * p[:, :2], axis=1)
    mesh = plsc.VectorSubcoreMesh(core_axis_name="c", subcore_axis_name="s")

    @functools.partial(
        pl.kernel,
        mesh=mesh,
        compiler_params=pltpu.CompilerParams(
            use_tc_tiling_on_sc=False, needs_layout_passes=False
        ),
        out_type=jax.ShapeDtypeStruct((n * _NCAND,), jnp.float32),
        scratch_types=[
            pltpu.VMEM((n,), jnp.float32),
            pltpu.VMEM((n,), jnp.float32),
            pltpu.VMEM((n,), jnp.float32),
            pltpu.VMEM((rg * _KCH,), jnp.int32),
            pltpu.VMEM((rg * _NCAND,), jnp.float32),
        ],
    )
    def sc_cand(x_hbm, y_hbm, sq_hbm, ch_hbm, out_hbm, xv, yv, sqv, chv, outv):
        wid = lax.axis_index("s") * nc + lax.axis_index("c")
        base = wid * rw
        pltpu.sync_copy(x_hbm, xv)
        pltpu.sync_copy(y_hbm, yv)
        pltpu.sync_copy(sq_hbm, sqv)

        def group_body(g, carry):
            r0 = base + g * rg
            pltpu.sync_copy(ch_hbm.at[pl.ds(r0 * _KCH, rg * _KCH)], chv)

            def row_body(r, carry2):
                rid = jnp.full((16,), r0 + r, jnp.int32)
                rx = plsc.load_gather(xv, [rid])
                ry = plsc.load_gather(yv, [rid])
                rsq = plsc.load_gather(sqv, [rid])
                rxb = _bf16_round_sc(rx)
                ryb = _bf16_round_sc(ry)
                for h in range(_KCH // 16):
                    cv = chv[pl.ds(r * _KCH + h * 16, 16)]
                    for a in range(_CA):
                        cand = cv + _NCH * a
                        gx = plsc.load_gather(xv, [cand])
                        gy = plsc.load_gather(yv, [cand])
                        gsq = plsc.load_gather(sqv, [cand])
                        gxb = _bf16_round_sc(gx)
                        gyb = _bf16_round_sc(gy)
                        cross = rxb * gxb + ryb * gyb
                        d2 = (rsq + gsq) - 2.0 * cross
                        outv[pl.ds(r * _NCAND + a * _KCH + h * 16, 16)] = d2
                return carry2

            lax.fori_loop(0, rg, row_body, 0)
            pltpu.sync_copy(
                outv, out_hbm.at[pl.ds(r0 * _NCAND, rg * _NCAND)]
            )
            return carry

        lax.fori_loop(0, ngr, group_body, 0)

    return sc_cand(x, y, sq, chunks.reshape(-1))


def _stage3_body(d2_ref, cand_ref, out_ref):
    i = pl.program_id(0)
    br = d2_ref.shape[0]
    n_total = _NCH * _CA
    neg = -d2_ref[...]
    candid = cand_ref[...]
    row_ids = i * br + lax.broadcasted_iota(jnp.int32, (br, 1), 0)
    neg = jnp.where(candid == row_ids, -jnp.inf, neg)

    sels = []
    for _ in range(_K):
        g = jnp.max(neg, axis=1, keepdims=True)
        sel = jnp.min(jnp.where(neg == g, candid, n_total),
                      axis=1, keepdims=True)
        sels.append(sel)
        neg = jnp.where(candid == sel, -jnp.inf, neg)
    out_ref[...] = jnp.concatenate(sels, axis=1)


def _topk_of_candidates(d2c, chunks):
    n = chunks.shape[0]
    grid = n // _BR3
    cand = (chunks[:, None, :]
            + _NCH * jnp.arange(_CA, dtype=jnp.int32)[:, None]
            ).reshape(n, _NCAND)
    return pl.pallas_call(
        _stage3_body,
        grid=(grid,),
        in_specs=[
            pl.BlockSpec((_BR3, _NCAND), lambda i: (i, 0)),
            pl.BlockSpec((_BR3, _NCAND), lambda i: (i, 0)),
        ],
        out_specs=pl.BlockSpec((_BR3, _K), lambda i: (i, 0)),
        out_shape=jax.ShapeDtypeStruct((n, _K), jnp.int32),
        compiler_params=pltpu.CompilerParams(
            dimension_semantics=("parallel",),
        ),
    )(d2c.reshape(n, _NCAND), cand)


def _disp_gather(p, idx):
    n, k = idx.shape
    info = plsc.get_sparse_core_info()
    nc = info.num_cores
    nw = nc * info.num_subcores
    rw = n // nw
    x = p[:, 0]
    y = p[:, 1]
    mesh = plsc.VectorSubcoreMesh(core_axis_name="c", subcore_axis_name="s")

    @functools.partial(
        pl.kernel,
        mesh=mesh,
        compiler_params=pltpu.CompilerParams(
            use_tc_tiling_on_sc=False, needs_layout_passes=False
        ),
        out_type=jax.ShapeDtypeStruct((n * k * 2,), jnp.float32),
        scratch_types=[
            pltpu.VMEM((n,), jnp.float32),
            pltpu.VMEM((n,), jnp.float32),
            pltpu.VMEM((rw * k,), jnp.int32),
            pltpu.VMEM((rw * k * 2,), jnp.float32),
        ],
    )
    def sc_gather(x_hbm, y_hbm, idx_hbm, out_hbm, xv, yv, idxv, outv):
        wid = lax.axis_index("s") * nc + lax.axis_index("c")
        base = wid * rw
        pltpu.sync_copy(x_hbm, xv)
        pltpu.sync_copy(y_hbm, yv)
        pltpu.sync_copy(idx_hbm.at[pl.ds(base * k, rw * k)], idxv)
        iota16 = lax.iota(jnp.int32, 16)

        def row_body(r, carry):
            rvec = jnp.full((16,), base + r, jnp.int32)
            rx = plsc.load_gather(xv, [rvec])
            ry = plsc.load_gather(yv, [rvec])
            for v in range(k // 16):
                iv = idxv[pl.ds(r * k + v * 16, 16)]
                gx = plsc.load_gather(xv, [iv])
                gy = plsc.load_gather(yv, [iv])
                pos = r * (k * 2) + v * 32 + 2 * iota16
                plsc.store_scatter(outv, [pos], rx - gx)
                plsc.store_scatter(outv, [pos + 1], ry - gy)
            return carry

        lax.fori_loop(0, rw, row_body, 0)
        pltpu.sync_copy(outv, out_hbm.at[pl.ds(base * k * 2, rw * k * 2)])

    return sc_gather(x, y, idx.reshape(-1)).reshape(n, k, 2)


def kernel(p):
    chunks = _top_chunks(p)
    d2c = _stage2_cand(p, chunks)
    idx = _topk_of_candidates(d2c, chunks)
    return _disp_gather(p, idx)


# speedup vs baseline: 3.1633x; 1.0269x over previous
import functools

import jax
import jax.numpy as jnp
from jax import lax
from jax.experimental import pallas as pl
from jax.experimental.pallas import tpu as pltpu
from jax.experimental.pallas import tpu_sc as plsc

_K = 32
_BR = 256
_BR3 = 512
_NCH = 512
_CA = 32
_KCH = 48
_NCAND = _KCH * _CA


def _stage1_body(xr_ref, yr_ref, sqr_ref, xc_ref, yc_ref, sqc_ref, out_ref):
    n = xc_ref.shape[1]
    xrb = xr_ref[...].astype(jnp.bfloat16)
    yrb = yr_ref[...].astype(jnp.bfloat16)
    xcb = xc_ref[...].astype(jnp.bfloat16)
    ycb = yc_ref[...].astype(jnp.bfloat16)
    cross = jax.lax.dot_general(
        jnp.concatenate([xrb, yrb], axis=1),
        jnp.concatenate([xcb, ycb], axis=0),
        (((1,), (0,)), ((), ())),
        preferred_element_type=jnp.float32,
    )
    d2 = (sqr_ref[...] + sqc_ref[...]) - 2.0 * cross

    dm = d2[:, 0:_NCH]
    for a in range(1, _CA):
        dm = jnp.minimum(dm, d2[:, a * _NCH:(a + 1) * _NCH])

    chid = lax.broadcasted_iota(jnp.int32, (1, _NCH), 1)
    sels = []
    for _ in range(_KCH):
        g = jnp.min(dm, axis=1, keepdims=True)
        sel = jnp.min(jnp.where(dm == g, chid, _NCH), axis=1, keepdims=True)
        sels.append(sel)
        dm = jnp.where(chid == sel, jnp.inf, dm)
    out_ref[...] = jnp.concatenate(sels, axis=1)


def _top_chunks(p):
    n = p.shape[0]
    x = p[:, 0]
    y = p[:, 1]
    sq = jnp.sum(p[:, :2] * p[:, :2], axis=1)
    grid = n // _BR
    row_spec = pl.BlockSpec((_BR, 1), lambda i: (i, 0))
    col_spec = pl.BlockSpec((1, n), lambda i: (0, 0))
    return pl.pallas_call(
        _stage1_body,
        grid=(grid,),
        in_specs=[row_spec, row_spec, row_spec, col_spec, col_spec, col_spec],
        out_specs=pl.BlockSpec((_BR, _KCH), lambda i: (i, 0)),
        out_shape=jax.ShapeDtypeStruct((n, _KCH), jnp.int32),
        compiler_params=pltpu.CompilerParams(
            dimension_semantics=("parallel",),
        ),
    )(x.reshape(n, 1), y.reshape(n, 1), sq.reshape(n, 1),
      x.reshape(1, n), y.reshape(1, n), sq.reshape(1, n))


def _bf16_round_sc(v):
    bits = plsc.bitcast(v, jnp.int32)
    r = (bits + 0x7FFF + ((bits >> 16) & 1)) & jnp.int32(-65536)
    return plsc.bitcast(r, jnp.float32)


def _stage2_cand(p, chunks):
    n = p.shape[0]
    info = plsc.get_sparse_core_info()
    nc = info.num_cores
    nw = nc * info.num_subcores
    rw = n // nw
    rg = 32
    ngr = rw // rg
    x = p[:, 0]
    y = p[:, 1]
    sq = jnp.sum(p[:, :2] * p[:, :2], axis=1)
    mesh = plsc.VectorSubcoreMesh(core_axis_name="c", subcore_axis_name="s")

    @functools.partial(
        pl.kernel,
        mesh=mesh,
        compiler_params=pltpu.CompilerParams(
            use_tc_tiling_on_sc=False, needs_layout_passes=False
        ),
        out_type=jax.ShapeDtypeStruct((n * _NCAND,), jnp.float32),
        scratch_types=[
            pltpu.VMEM((n,), jnp.float32),
            pltpu.VMEM((n,), jnp.float32),
            pltpu.VMEM((n,), jnp.float32),
            pltpu.VMEM((rg * _KCH,), jnp.int32),
            pltpu.VMEM((rg * _NCAND,), jnp.float32),
        ],
    )
    def sc_cand(x_hbm, y_hbm, sq_hbm, ch_hbm, out_hbm, xv, yv, sqv, chv, outv):
        wid = lax.axis_index("s") * nc + lax.axis_index("c")
        base = wid * rw
        pltpu.sync_copy(x_hbm, xv)
        pltpu.sync_copy(y_hbm, yv)
        pltpu.sync_copy(sq_hbm, sqv)

        def group_body(g, carry):
            r0 = base + g * rg
            pltpu.sync_copy(ch_hbm.at[pl.ds(r0 * _KCH, rg * _KCH)], chv)

            def row_body(r, carry2):
                rid = jnp.full((16,), r0 + r, jnp.int32)
                rx = plsc.load_gather(xv, [rid])
                ry = plsc.load_gather(yv, [rid])
                rsq = plsc.load_gather(sqv, [rid])
                rxb = _bf16_round_sc(rx)
                ryb = _bf16_round_sc(ry)
                for h in range(_KCH // 16):
                    cv = chv[pl.ds(r * _KCH + h * 16, 16)]
                    for a in range(_CA):
                        cand = cv + _NCH * a
                        gx = plsc.load_gather(xv, [cand])
                        gy = plsc.load_gather(yv, [cand])
                        gsq = plsc.load_gather(sqv, [cand])
                        gxb = _bf16_round_sc(gx)
                        gyb = _bf16_round_sc(gy)
                        cross = rxb * gxb + ryb * gyb
                        d2 = (rsq + gsq) - 2.0 * cross
                        outv[pl.ds(r * _NCAND + a * _KCH + h * 16, 16)] = d2
                return carry2

            lax.fori_loop(0, rg, row_body, 0)
            pltpu.sync_copy(
                outv, out_hbm.at[pl.ds(r0 * _NCAND, rg * _NCAND)]
            )
            return carry

        lax.fori_loop(0, ngr, group_body, 0)

    return sc_cand(x, y, sq, chunks.reshape(-1))


def _stage3_body(d2_ref, cand_ref, out_ref):
    i = pl.program_id(0)
    br = d2_ref.shape[0]
    n_total = _NCH * _CA
    neg = -d2_ref[...]
    candid = cand_ref[...]
    row_ids = i * br + lax.broadcasted_iota(jnp.int32, (br, 1), 0)
    neg = jnp.where(candid == row_ids, -jnp.inf, neg)

    sels = []
    for _ in range(_K):
        g = jnp.max(neg, axis=1, keepdims=True)
        sel = jnp.min(jnp.where(neg == g, candid, n_total),
                      axis=1, keepdims=True)
        sels.append(sel)
        neg = jnp.where(candid == sel, -jnp.inf, neg)
    out_ref[...] = jnp.concatenate(sels, axis=1)


def _topk_of_candidates(d2c, chunks):
    n = chunks.shape[0]
    grid = n // _BR3
    cand = (chunks[:, None, :]
            + _NCH * jnp.arange(_CA, dtype=jnp.int32)[:, None]
            ).reshape(n, _NCAND)
    return pl.pallas_call(
        _stage3_body,
        grid=(grid,),
        in_specs=[
            pl.BlockSpec((_BR3, _NCAND), lambda i: (i, 0)),
            pl.BlockSpec((_BR3, _NCAND), lambda i: (i, 0)),
        ],
        out_specs=pl.BlockSpec((_BR3, _K), lambda i: (i, 0)),
        out_shape=jax.ShapeDtypeStruct((n, _K), jnp.int32),
        compiler_params=pltpu.CompilerParams(
            dimension_semantics=("parallel",),
        ),
    )(d2c.reshape(n, _NCAND), cand)


def _disp_gather(p, idx):
    n, k = idx.shape
    info = plsc.get_sparse_core_info()
    nc = info.num_cores
    nw = nc * info.num_subcores
    rw = n // nw
    x = p[:, 0]
    y = p[:, 1]
    mesh = plsc.VectorSubcoreMesh(core_axis_name="c", subcore_axis_name="s")

    @functools.partial(
        pl.kernel,
        mesh=mesh,
        compiler_params=pltpu.CompilerParams(
            use_tc_tiling_on_sc=False, needs_layout_passes=False
        ),
        out_type=jax.ShapeDtypeStruct((n * k * 2,), jnp.float32),
        scratch_types=[
            pltpu.VMEM((n,), jnp.float32),
            pltpu.VMEM((n,), jnp.float32),
            pltpu.VMEM((rw * k,), jnp.int32),
            pltpu.VMEM((rw * k * 2,), jnp.float32),
        ],
    )
    def sc_gather(x_hbm, y_hbm, idx_hbm, out_hbm, xv, yv, idxv, outv):
        wid = lax.axis_index("s") * nc + lax.axis_index("c")
        base = wid * rw
        pltpu.sync_copy(x_hbm, xv)
        pltpu.sync_copy(y_hbm, yv)
        pltpu.sync_copy(idx_hbm.at[pl.ds(base * k, rw * k)], idxv)
        iota16 = lax.iota(jnp.int32, 16)

        def row_body(r, carry):
            rvec = jnp.full((16,), base + r, jnp.int32)
            rx = plsc.load_gather(xv, [rvec])
            ry = plsc.load_gather(yv, [rvec])
            for v in range(k // 16):
                iv = idxv[pl.ds(r * k + v * 16, 16)]
                gx = plsc.load_gather(xv, [iv])
                gy = plsc.load_gather(yv, [iv])
                pos = r * (k * 2) + v * 32 + 2 * iota16
                plsc.store_scatter(outv, [pos], rx - gx)
                plsc.store_scatter(outv, [pos + 1], ry - gy)
            return carry

        lax.fori_loop(0, rw, row_body, 0)
        pltpu.sync_copy(outv, out_hbm.at[pl.ds(base * k * 2, rw * k * 2)])

    return sc_gather(x, y, idx.reshape(-1)).reshape(n, k, 2)


def kernel(p):
    chunks = _top_chunks(p)
    d2c = _stage2_cand(p, chunks)
    idx = _topk_of_candidates(d2c, chunks)
    return _disp_gather(p, idx)
